# baseline (speedup 1.0000x reference)
import functools

import jax
import jax.numpy as jnp
from jax import lax
from jax.experimental import pallas as pl
from jax.experimental.pallas import tpu as pltpu
from jax.experimental.pallas import tpu_sc as plsc

_D = 64
_B = 4096
_MAX_NORM = 1.0
_L = 16

_NC = 2
_NS = 16
_NW = _NC * _NS
_HD = _D // 2
_BPP = _B // (_NW // 2)
_NBUF = 8
_G = 4 * _L


def _make_sc_call():
    mesh = plsc.VectorSubcoreMesh(core_axis_name="c", subcore_axis_name="s",
                                  num_cores=_NC, num_subcores=_NS)
    out_sds = jax.ShapeDtypeStruct((_D, _B), jnp.float32)

    @functools.partial(
        pl.kernel,
        out_type=(out_sds, out_sds, out_sds),
        mesh=mesh,
        compiler_params=pltpu.CompilerParams(needs_layout_passes=False),
        scratch_types=(
            [pltpu.VMEM((_BPP,), jnp.int32)]
            + [pltpu.VMEM((_HD, 128), jnp.float32)] * _NBUF
            + [pltpu.VMEM((_HD, _BPP), jnp.float32)]
            + [pltpu.VMEM((_BPP,), jnp.float32)] * 2
            + [pltpu.VMEM_SHARED((_NS, _BPP), jnp.float32)]
            + [pltpu.SemaphoreType.DMA] * (_NBUF + 1)
        ),
    )
    def sc_call(ig, ia, ie, wg, wa, we, og, oa, oe, idx_v, *rest):
        slabs = rest[0:_NBUF]
        outb = rest[_NBUF]
        myss_v = rest[_NBUF + 1]
        prtss_v = rest[_NBUF + 2]
        ssh = rest[_NBUF + 3]
        sems = rest[_NBUF + 4:2 * _NBUF + 4]
        xsem = rest[2 * _NBUF + 4]
        s = lax.axis_index("s")
        c = lax.axis_index("c")
        half = s % 2
        pair = (s // 2) * _NC + c
        base = pair * _BPP
        r0 = pl.multiple_of(half * _HD, 8)
        lane = lax.iota(jnp.int32, _L)

        def fetch(w_hbm, e, b):
            c128 = pl.multiple_of((e // 128) * 128, 128)
            return pltpu.async_copy(
                w_hbm.at[pl.ds(r0, _HD), pl.ds(c128, 128)],
                slabs[b], sems[b])

        def renorm_write(out_hbm):
            def p1(g, _):
                c0 = g * _L
                ss = jnp.zeros((_L,), jnp.float32)
                for j in range(_HD):
                    v = outb[j, pl.ds(c0, _L)]
                    ss = ss + v * v
                myss_v[pl.ds(c0, _L)] = ss
                return ()
            lax.fori_loop(0, _BPP // _L, p1, ())
            pltpu.sync_copy(myss_v, ssh.at[s])
            plsc.subcore_barrier()
            pltpu.sync_copy(ssh.at[s + 1 - 2 * half], prtss_v)
            plsc.subcore_barrier()

            def p2(g, _):
                c0 = g * _L
                ss = myss_v[pl.ds(c0, _L)] + prtss_v[pl.ds(c0, _L)]
                ssc = jnp.maximum(ss, 1.0)
                i = plsc.bitcast(ssc, jnp.int32)
                i = jnp.int32(0x5F3759DF) - (i >> 1)
                y = plsc.bitcast(i, jnp.float32)
                for _u in range(3):
                    y = y * (1.5 - 0.5 * ssc * y * y)
                norm = ssc * y
                scale = jnp.where(ss > _MAX_NORM * _MAX_NORM,
                                  _MAX_NORM / (norm + 1e-7), 1.0)
                for j in range(_HD):
                    outb[j, pl.ds(c0, _L)] = outb[j, pl.ds(c0, _L)] * scale
                return ()
            lax.fori_loop(0, _BPP // _L, p2, ())
            pltpu.sync_copy(
                outb, out_hbm.at[pl.ds(r0, _HD), pl.ds(base, _BPP)])

        for idx_hbm, w_hbm, out_hbm in ((ig, wg, og), (ia, wa, oa),
                                        (ie, we, oe)):
            pltpu.sync_copy(idx_hbm.at[pl.ds(base, _BPP)], idx_v)

            def body(g, _):
                ev = [idx_v[pl.ds(g * _G + i * _L, _L)]
                      for i in range(_G // _L)]

                def e_at(t):
                    return ev[t // _L][t % _L]

                copies = [None] * _NBUF
                for t in range(_NBUF - 1):
                    copies[t] = fetch(w_hbm, e_at(t), t)
                for t in range(_G):
                    if t + _NBUF - 1 < _G:
                        b = (t + _NBUF - 1) % _NBUF
                        copies[b] = fetch(w_hbm, e_at(t + _NBUF - 1), b)
                    copies[t % _NBUF].wait()
                    e = e_at(t)
                    col = jnp.full((_L,), e % 128, jnp.int32)
                    kvec = jnp.full((_L,), g * _G + t, jnp.int32)
                    for jj in range(_HD // _L):
                        rows = jj * _L + lane
                        v = plsc.load_gather(slabs[t % _NBUF], [rows, col])
                        plsc.store_scatter(outb, [rows, kvec], v)
                return ()
            lax.fori_loop(0, _BPP // _G, body, ())

            renorm_write(out_hbm)

    return sc_call


def kernel(latent_idx_geo, latent_idx_app, latent_idx_exp, W_geo, W_app,
           W_exp):
    ig = latent_idx_geo.astype(jnp.int32)
    ia = latent_idx_app.astype(jnp.int32)
    ie = latent_idx_exp.astype(jnp.int32)
    call = _make_sc_call()
    og, oa, oe = call(ig, ia, ie, W_geo.T, W_app.T, W_exp.T)
    return (og.T, oa.T, oe.T)


# speedup vs baseline: 1.0777x; 1.0777x over previous
import functools

import jax
import jax.numpy as jnp
from jax import lax
from jax.experimental import pallas as pl
from jax.experimental.pallas import tpu as pltpu
from jax.experimental.pallas import tpu_sc as plsc

_D = 64
_B = 4096
_MAX_NORM = 1.0
_L = 16

_NC = 2
_NS = 16
_NW = _NC * _NS
_BPW = _B // _NW


def _renorm_blk(blk):
    def grp(g, _):
        c0 = g * _L
        ss = jnp.zeros((_L,), jnp.float32)
        for j in range(_D):
            v = blk[j, pl.ds(c0, _L)]
            ss = ss + v * v
        ssc = jnp.maximum(ss, 1.0)
        i = plsc.bitcast(ssc, jnp.int32)
        i = jnp.int32(0x5F3759DF) - (i >> 1)
        y = plsc.bitcast(i, jnp.float32)
        for _ in range(3):
            y = y * (1.5 - 0.5 * ssc * y * y)
        norm = ssc * y
        scale = jnp.where(ss > _MAX_NORM * _MAX_NORM,
                          _MAX_NORM / (norm + 1e-7), 1.0)
        for j in range(_D):
            blk[j, pl.ds(c0, _L)] = blk[j, pl.ds(c0, _L)] * scale
        return ()
    lax.fori_loop(0, _BPW // _L, grp, ())


def _make_sc_call():
    mesh = plsc.VectorSubcoreMesh(core_axis_name="c", subcore_axis_name="s",
                                  num_cores=_NC, num_subcores=_NS)
    out_sds = jax.ShapeDtypeStruct((_D, _B), jnp.float32)

    @functools.partial(
        pl.kernel,
        out_type=(out_sds, out_sds, out_sds),
        mesh=mesh,
        compiler_params=pltpu.CompilerParams(needs_layout_passes=False),
        scratch_types=(
            [pltpu.VMEM((_BPW,), jnp.int32)]
            + [pltpu.VMEM((_D, 128), jnp.float32)] * 12
            + [pltpu.VMEM((_D, _BPW), jnp.float32)]
            + [pltpu.SemaphoreType.DMA] * 12
        ),
    )
    def sc_call(ig, ia, ie, wg, wa, we, og, oa, oe, idx_v, *rest):
        slabs = rest[0:12]
        outb = rest[12]
        sems = rest[13:25]
        wid = lax.axis_index("s") * _NC + lax.axis_index("c")
        base = wid * _BPW
        lane = lax.iota(jnp.int32, _L)
        _G = 4 * _L
        _NBUF = 12

        def fetch(e, b):
            c128 = pl.multiple_of((e // 128) * 128, 128)
            return pltpu.async_copy(w_hbm.at[:, pl.ds(c128, 128)],
                                    slabs[b], sems[b])

        for idx_hbm, w_hbm, out_hbm in ((ig, wg, og), (ia, wa, oa),
                                        (ie, we, oe)):
            pltpu.sync_copy(idx_hbm.at[pl.ds(base, _BPW)], idx_v)

            def body(g, _):
                ev = [idx_v[pl.ds(g * _G + i * _L, _L)]
                      for i in range(_G // _L)]

                def e_at(t):
                    return ev[t // _L][t % _L]

                copies = [None] * _NBUF
                for t in range(_NBUF - 1):
                    copies[t] = fetch(e_at(t), t)
                for t in range(_G):
                    if t + _NBUF - 1 < _G:
                        b = (t + _NBUF - 1) % _NBUF
                        copies[b] = fetch(e_at(t + _NBUF - 1), b)
                    copies[t % _NBUF].wait()
                    e = e_at(t)
                    col = jnp.full((_L,), e % 128, jnp.int32)
                    kvec = jnp.full((_L,), g * _G + t, jnp.int32)
                    for jj in range(_D // _L):
                        rows = jj * _L + lane
                        v = plsc.load_gather(slabs[t % _NBUF], [rows, col])
                        plsc.store_scatter(outb, [rows, kvec], v)
                return ()
            lax.fori_loop(0, _BPW // _G, body, ())

            _renorm_blk(outb)
            pltpu.sync_copy(outb, out_hbm.at[:, pl.ds(base, _BPW)])

    return sc_call


def kernel(latent_idx_geo, latent_idx_app, latent_idx_exp, W_geo, W_app,
           W_exp):
    ig = latent_idx_geo.astype(jnp.int32)
    ia = latent_idx_app.astype(jnp.int32)
    ie = latent_idx_exp.astype(jnp.int32)
    call = _make_sc_call()
    og, oa, oe = call(ig, ia, ie, W_geo.T, W_app.T, W_exp.T)
    return (og.T, oa.T, oe.T)


# speedup vs baseline: 1.0953x; 1.0163x over previous
import functools

import jax
import jax.numpy as jnp
from jax import lax
from jax.experimental import pallas as pl
from jax.experimental.pallas import tpu as pltpu
from jax.experimental.pallas import tpu_sc as plsc

_D = 64
_B = 4096
_MAX_NORM = 1.0
_L = 16

_NC = 2
_NS = 16
_NW = _NC * _NS
_BPW = _B // _NW


def _renorm_blk(blk):
    def grp(g, _):
        c0 = g * _L
        ss = jnp.zeros((_L,), jnp.float32)
        for j in range(_D):
            v = blk[j, pl.ds(c0, _L)]
            ss = ss + v * v
        ssc = jnp.maximum(ss, 1.0)
        i = plsc.bitcast(ssc, jnp.int32)
        i = jnp.int32(0x5F3759DF) - (i >> 1)
        y = plsc.bitcast(i, jnp.float32)
        for _ in range(3):
            y = y * (1.5 - 0.5 * ssc * y * y)
        norm = ssc * y
        scale = jnp.where(ss > _MAX_NORM * _MAX_NORM,
                          _MAX_NORM / (norm + 1e-7), 1.0)
        for j in range(_D):
            blk[j, pl.ds(c0, _L)] = blk[j, pl.ds(c0, _L)] * scale
        return ()
    lax.fori_loop(0, _BPW // _L, grp, ())


def _make_sc_call():
    mesh = plsc.VectorSubcoreMesh(core_axis_name="c", subcore_axis_name="s",
                                  num_cores=_NC, num_subcores=_NS)
    out_sds = jax.ShapeDtypeStruct((_D, _B), jnp.float32)

    @functools.partial(
        pl.kernel,
        out_type=(out_sds, out_sds, out_sds),
        mesh=mesh,
        compiler_params=pltpu.CompilerParams(needs_layout_passes=False),
        scratch_types=(
            [pltpu.VMEM((_BPW,), jnp.int32)]
            + [pltpu.VMEM((_D, 128), jnp.float32)] * 12
            + [pltpu.VMEM((_D, _BPW), jnp.float32)]
            + [pltpu.SemaphoreType.DMA] * 12
        ),
    )
    def sc_call(ig, ia, ie, wg, wa, we, og, oa, oe, idx_v, *rest):
        slabs = rest[0:12]
        outb = rest[12]
        sems = rest[13:25]
        wid = lax.axis_index("s") * _NC + lax.axis_index("c")
        base = wid * _BPW
        lane = lax.iota(jnp.int32, _L)
        _G = 4 * _L
        _NBUF = 8

        def fetch(e, b):
            c128 = pl.multiple_of((e // 128) * 128, 128)
            return pltpu.async_copy(w_hbm.at[:, pl.ds(c128, 128)],
                                    slabs[b], sems[b])

        for idx_hbm, w_hbm, out_hbm in ((ig, wg, og), (ia, wa, oa),
                                        (ie, we, oe)):
            pltpu.sync_copy(idx_hbm.at[pl.ds(base, _BPW)], idx_v)

            def body(g, _):
                ev = [idx_v[pl.ds(g * _G + i * _L, _L)]
                      for i in range(_G // _L)]

                def e_at(t):
                    return ev[t // _L][t % _L]

                copies = [None] * _NBUF
                for t in range(_NBUF - 1):
                    copies[t] = fetch(e_at(t), t)
                for t in range(_G):
                    if t + _NBUF - 1 < _G:
                        b = (t + _NBUF - 1) % _NBUF
                        copies[b] = fetch(e_at(t + _NBUF - 1), b)
                    copies[t % _NBUF].wait()
                    e = e_at(t)
                    col = jnp.full((_L,), e % 128, jnp.int32)
                    kvec = jnp.full((_L,), g * _G + t, jnp.int32)
                    for jj in range(_D // _L):
                        rows = jj * _L + lane
                        v = plsc.load_gather(slabs[t % _NBUF], [rows, col])
                        plsc.store_scatter(outb, [rows, kvec], v)
                return ()
            lax.fori_loop(0, _BPW // _G, body, ())

            _renorm_blk(outb)
            pltpu.sync_copy(outb, out_hbm.at[:, pl.ds(base, _BPW)])

    return sc_call


def kernel(latent_idx_geo, latent_idx_app, latent_idx_exp, W_geo, W_app,
           W_exp):
    ig = latent_idx_geo.astype(jnp.int32)
    ia = latent_idx_app.astype(jnp.int32)
    ie = latent_idx_exp.astype(jnp.int32)
    call = _make_sc_call()
    og, oa, oe = call(ig, ia, ie, W_geo.T, W_app.T, W_exp.T)
    return (og.T, oa.T, oe.T)
